# Initial kernel scaffold; baseline (speedup 1.0000x reference)
import jax
import jax.numpy as jnp
from jax.experimental import pallas as pl


def kernel(x, edge_index, batch_index, W0, b0, W1, b1, W2, b2, W3, b3, fc1_W, fc1_b, fc2_W, fc2_b, out_W, out_b):
    raise NotImplementedError("write your pallas kernel here")


# speedup vs baseline: 10.2250x; 10.2250x over previous
import functools

import jax
import jax.numpy as jnp
from jax import lax
from jax.experimental import pallas as pl
from jax.experimental.pallas import tpu as pltpu
from jax.experimental.pallas import tpu_sc as plsc

N = 10000
E = 320000
F = 128
G = 16
NPAD = 10240
BLK = 1024
NC, NS = 2, 16
NW = NC * NS
PER_W = E // NW
CH = 80
STEPS = PER_W // CH
RPT = NPAD // NS
ZR = 128
DEGW = 16

_mesh = plsc.VectorSubcoreMesh(core_axis_name="c", subcore_axis_name="s")


@functools.partial(
    pl.kernel,
    out_type=jax.ShapeDtypeStruct((NC, NPAD, DEGW), jnp.float32),
    mesh=_mesh,
    scratch_types=[
        pltpu.VMEM((CH,), jnp.int32),
        pltpu.VMEM((CH, DEGW), jnp.float32),
        pltpu.VMEM((ZR, DEGW), jnp.float32),
        pltpu.VMEM_SHARED((NPAD, DEGW), jnp.float32),
    ],
)
def _sc_degree(dst_hbm, out_hbm, didx, ones_v, zbuf, deg_sh):
    c = lax.axis_index("c")
    s = lax.axis_index("s")

    def fill(r, _):
        zbuf[r, pl.ds(0, 16)] = jnp.zeros((16,), jnp.float32)
        return 0
    lax.fori_loop(0, ZR, fill, 0)

    def fill1(r, _):
        ones_v[r, pl.ds(0, 16)] = jnp.ones((16,), jnp.float32)
        return 0
    lax.fori_loop(0, CH, fill1, 0)

    r0 = s * RPT
    for zi in range(RPT // ZR):
        pltpu.sync_copy(zbuf, deg_sh.at[pl.ds(r0 + zi * ZR, ZR)])
    plsc.subcore_barrier()

    base = (c * NS + s) * PER_W

    def step(j, _):
        off = pl.multiple_of(base + j * CH, 8)
        pltpu.sync_copy(dst_hbm.at[pl.ds(off, CH)], didx)
        pltpu.sync_copy(ones_v, deg_sh.at[didx], add=True)
        return 0
    lax.fori_loop(0, STEPS, step, 0)

    plsc.subcore_barrier()
    pltpu.sync_copy(deg_sh.at[pl.ds(r0, RPT)], out_hbm.at[c, pl.ds(r0, RPT)])


@functools.partial(
    pl.kernel,
    out_type=jax.ShapeDtypeStruct((NC, NPAD, F), jnp.float32),
    mesh=_mesh,
    scratch_types=[
        pltpu.VMEM((CH,), jnp.int32),
        pltpu.VMEM((CH,), jnp.int32),
        pltpu.VMEM((CH, F), jnp.float32),
        pltpu.VMEM((ZR, F), jnp.float32),
        pltpu.VMEM_SHARED((NPAD, F), jnp.float32),
        pltpu.SemaphoreType.DMA,
    ],
)
def _sc_aggregate(hp_hbm, src_hbm, dst_hbm, out_hbm,
                  sidx, didx, rows, zbuf, agg_sh, sem):
    c = lax.axis_index("c")
    s = lax.axis_index("s")

    def fill(r, _):
        for i in range(F // 16):
            zbuf[r, pl.ds(i * 16, 16)] = jnp.zeros((16,), jnp.float32)
        return 0
    lax.fori_loop(0, ZR, fill, 0)

    r0 = s * RPT
    for zi in range(RPT // ZR):
        pltpu.sync_copy(zbuf, agg_sh.at[pl.ds(r0 + zi * ZR, ZR)])
    plsc.subcore_barrier()

    base = (c * NS + s) * PER_W

    def step(j, _):
        off = pl.multiple_of(base + j * CH, 8)
        pltpu.sync_copy(src_hbm.at[pl.ds(off, CH)], sidx)
        pltpu.sync_copy(dst_hbm.at[pl.ds(off, CH)], didx)
        pltpu.async_copy(hp_hbm.at[sidx], rows, sem).wait()
        pltpu.sync_copy(rows, agg_sh.at[didx], add=True)
        return 0
    lax.fori_loop(0, STEPS, step, 0)

    plsc.subcore_barrier()
    pltpu.sync_copy(agg_sh.at[pl.ds(r0, RPT)], out_hbm.at[c, pl.ds(r0, RPT)])


def _prep_body(deg_ref, x_ref, w_ref, dinv_ref, hp_ref):
    deg = deg_ref[0, :, 0:1] + deg_ref[1, :, 0:1] + 1.0
    dinv = lax.rsqrt(deg)
    dinv_ref[...] = dinv
    hp_ref[...] = jnp.dot(x_ref[...], w_ref[...],
                          preferred_element_type=jnp.float32) * dinv


def _tc_prep(deg2, x, w):
    grid = NPAD // BLK
    return pl.pallas_call(
        _prep_body,
        grid=(grid,),
        in_specs=[
            pl.BlockSpec((NC, BLK, DEGW), lambda i: (0, i, 0)),
            pl.BlockSpec((BLK, F), lambda i: (i, 0)),
            pl.BlockSpec((F, F), lambda i: (0, 0)),
        ],
        out_specs=[
            pl.BlockSpec((BLK, 1), lambda i: (i, 0)),
            pl.BlockSpec((BLK, F), lambda i: (i, 0)),
        ],
        out_shape=[
            jax.ShapeDtypeStruct((NPAD, 1), jnp.float32),
            jax.ShapeDtypeStruct((NPAD, F), jnp.float32),
        ],
    )(deg2, x, w)


def _layer_body(a2_ref, hp_ref, dinv_ref, b_ref, w_ref, out_ref):
    dinv = dinv_ref[...]
    conv = (a2_ref[0] + a2_ref[1] + hp_ref[...]) * dinv + b_ref[...]
    h = jnp.tanh(conv)
    out_ref[...] = jnp.dot(h, w_ref[...],
                           preferred_element_type=jnp.float32) * dinv


def _tc_layer(agg2, hp, dinv, b, w):
    grid = NPAD // BLK
    return pl.pallas_call(
        _layer_body,
        grid=(grid,),
        in_specs=[
            pl.BlockSpec((NC, BLK, F), lambda i: (0, i, 0)),
            pl.BlockSpec((BLK, F), lambda i: (i, 0)),
            pl.BlockSpec((BLK, 1), lambda i: (i, 0)),
            pl.BlockSpec((1, F), lambda i: (0, 0)),
            pl.BlockSpec((F, F), lambda i: (0, 0)),
        ],
        out_specs=pl.BlockSpec((BLK, F), lambda i: (i, 0)),
        out_shape=jax.ShapeDtypeStruct((NPAD, F), jnp.float32),
    )(agg2, hp, dinv, b.reshape(1, F), w)


def _pool_body(a2_ref, hp_ref, dinv_ref, b_ref, bid_ref, z_ref,
               gmp_acc, gap_acc, cnt_acc):
    i = pl.program_id(0)

    @pl.when(i == 0)
    def _init():
        gmp_acc[...] = jnp.full((G, F), -jnp.inf, jnp.float32)
        gap_acc[...] = jnp.zeros((G, F), jnp.float32)
        cnt_acc[...] = jnp.zeros((G, F), jnp.float32)

    conv = (a2_ref[0] + a2_ref[1] + hp_ref[...]) * dinv_ref[...] + b_ref[...]
    h = jnp.tanh(conv)
    bid = bid_ref[...]
    gids = lax.broadcasted_iota(jnp.int32, (1, G), 1)
    mask = (bid == gids).astype(jnp.float32)
    gap_acc[...] += lax.dot_general(
        mask, h, (((0,), (0,)), ((), ())), preferred_element_type=jnp.float32)
    cnt_acc[...] += lax.dot_general(
        mask, jnp.ones_like(h), (((0,), (0,)), ((), ())),
        preferred_element_type=jnp.float32)
    for g in range(G):
        rows = jnp.where(bid == g, h, -jnp.inf)
        gmp_acc[g:g + 1, :] = jnp.maximum(gmp_acc[g:g + 1, :],
                                          jnp.max(rows, axis=0, keepdims=True))

    @pl.when(i == pl.num_programs(0) - 1)
    def _fin():
        gap = gap_acc[...] / jnp.maximum(cnt_acc[...], 1.0)
        z_ref[...] = jnp.concatenate([gmp_acc[...], gap], axis=1)


def _tc_pool(agg2, hp, dinv, b, bid):
    grid = NPAD // BLK
    return pl.pallas_call(
        _pool_body,
        grid=(grid,),
        in_specs=[
            pl.BlockSpec((NC, BLK, F), lambda i: (0, i, 0)),
            pl.BlockSpec((BLK, F), lambda i: (i, 0)),
            pl.BlockSpec((BLK, 1), lambda i: (i, 0)),
            pl.BlockSpec((1, F), lambda i: (0, 0)),
            pl.BlockSpec((BLK, 1), lambda i: (i, 0)),
        ],
        out_specs=pl.BlockSpec((G, 2 * F), lambda i: (0, 0)),
        out_shape=jax.ShapeDtypeStruct((G, 2 * F), jnp.float32),
        scratch_shapes=[
            pltpu.VMEM((G, F), jnp.float32),
            pltpu.VMEM((G, F), jnp.float32),
            pltpu.VMEM((G, F), jnp.float32),
        ],
    )(agg2, hp, dinv, b.reshape(1, F), bid)


def _head_body(z_ref, w1_ref, b1_ref, w2_ref, b2_ref, w3_ref, b3_ref, o_ref):
    z = jax.nn.relu(jnp.dot(z_ref[...], w1_ref[...],
                            preferred_element_type=jnp.float32) + b1_ref[...])
    z = jax.nn.relu(jnp.dot(z, w2_ref[...],
                            preferred_element_type=jnp.float32) + b2_ref[...])
    o_ref[...] = jnp.dot(z, w3_ref[...],
                         preferred_element_type=jnp.float32) + b3_ref[...]


def _tc_head(z, fc1_W, fc1_b, fc2_W, fc2_b, outW_pad, outb_pad):
    return pl.pallas_call(
        _head_body,
        out_shape=jax.ShapeDtypeStruct((G, F), jnp.float32),
    )(z, fc1_W, fc1_b.reshape(1, -1), fc2_W, fc2_b.reshape(1, -1),
      outW_pad, outb_pad)


def kernel(x, edge_index, batch_index, W0, b0, W1, b1, W2, b2, W3, b3,
           fc1_W, fc1_b, fc2_W, fc2_b, out_W, out_b):
    src = edge_index[0]
    dst = edge_index[1]
    x_pad = jnp.pad(x, ((0, NPAD - N), (0, 0)))
    bid_pad = jnp.pad(batch_index, (0, NPAD - N),
                      constant_values=G).reshape(NPAD, 1)
    outW_pad = jnp.pad(out_W, ((0, 0), (0, F - out_W.shape[1])))
    outb_pad = jnp.pad(out_b, (0, F - out_b.shape[0])).reshape(1, F)

    deg2 = _sc_degree(dst)
    dinv, hp = _tc_prep(deg2, x_pad, W0)

    for b, w in ((b0, W1), (b1, W2), (b2, W3)):
        agg2 = _sc_aggregate(hp, src, dst)
        hp = _tc_layer(agg2, hp, dinv, b, w)

    agg2 = _sc_aggregate(hp, src, dst)
    z = _tc_pool(agg2, hp, dinv, b3, bid_pad)
    out = _tc_head(z, fc1_W, fc1_b, fc2_W, fc2_b, outW_pad, outb_pad)
    return out[:, :out_W.shape[1]]


# speedup vs baseline: 28.1462x; 2.7527x over previous
import functools

import jax
import jax.numpy as jnp
from jax import lax
from jax.experimental import pallas as pl
from jax.experimental.pallas import tpu as pltpu
from jax.experimental.pallas import tpu_sc as plsc

N = 10000
E = 320000
F = 128
G = 16
NPAD = 10240
BLK = 1024
NC, NS = 2, 16
NW = NC * NS
PER_W = E // NW
CH = 40
STEPS = PER_W // CH
RPT = NPAD // NS
ZR = 64
DEGW = 16
NBUF = 5
RING = 10
OUTER = STEPS // RING

_mesh = plsc.VectorSubcoreMesh(core_axis_name="c", subcore_axis_name="s")


@functools.partial(
    pl.kernel,
    out_type=jax.ShapeDtypeStruct((NC, NPAD, DEGW), jnp.float32),
    mesh=_mesh,
    scratch_types=[
        pltpu.VMEM((STEPS, CH), jnp.int32),
        pltpu.VMEM((CH, DEGW), jnp.float32),
        pltpu.VMEM((ZR, DEGW), jnp.float32),
        pltpu.VMEM_SHARED((NPAD, DEGW), jnp.float32),
    ],
)
def _sc_degree(dst3_hbm, out_hbm, didx, ones_v, zbuf, deg_sh):
    c = lax.axis_index("c")
    s = lax.axis_index("s")
    w = c * NS + s

    def fill(r, _):
        zbuf[r, pl.ds(0, 16)] = jnp.zeros((16,), jnp.float32)
        return 0
    lax.fori_loop(0, ZR, fill, 0)

    def fill1(r, _):
        ones_v[r, pl.ds(0, 16)] = jnp.ones((16,), jnp.float32)
        return 0
    lax.fori_loop(0, CH, fill1, 0)

    pltpu.sync_copy(dst3_hbm.at[w], didx)
    r0 = s * RPT
    for zi in range(RPT // ZR):
        pltpu.sync_copy(zbuf, deg_sh.at[pl.ds(r0 + zi * ZR, ZR)])
    plsc.subcore_barrier()

    def step(j, _):
        pltpu.sync_copy(ones_v, deg_sh.at[didx.at[j]], add=True)
        return 0
    lax.fori_loop(0, STEPS, step, 0)

    plsc.subcore_barrier()
    pltpu.sync_copy(deg_sh.at[pl.ds(r0, RPT)], out_hbm.at[c, pl.ds(r0, RPT)])


@functools.partial(
    pl.kernel,
    out_type=jax.ShapeDtypeStruct((NC, NPAD, F), jnp.float32),
    mesh=_mesh,
    scratch_types=[
        pltpu.VMEM((RING, CH), jnp.int32),
        pltpu.VMEM((RING, CH), jnp.int32),
        pltpu.VMEM((NBUF, CH, F), jnp.float32),
        pltpu.VMEM((ZR, F), jnp.float32),
        pltpu.VMEM_SHARED((NPAD, F), jnp.float32),
    ] + [pltpu.SemaphoreType.DMA] * (NBUF + RING),
)
def _sc_aggregate(hp_hbm, src3_hbm, dst3_hbm, out_hbm,
                  sidx, didx, rows, zbuf, agg_sh, *sems):
    sem_g = sems[:NBUF]
    sem_i = sems[NBUF:]
    c = lax.axis_index("c")
    s = lax.axis_index("s")
    w = c * NS + s

    def fill(r, _):
        for i in range(F // 16):
            zbuf[r, pl.ds(i * 16, 16)] = jnp.zeros((16,), jnp.float32)
        return 0
    lax.fori_loop(0, ZR, fill, 0)

    for q in range(RING):
        pltpu.async_copy(src3_hbm.at[w, q], sidx.at[q], sem_i[q])
        pltpu.async_copy(dst3_hbm.at[w, q], didx.at[q], sem_i[q])
    r0 = s * RPT
    for zi in range(RPT // ZR):
        pltpu.sync_copy(zbuf, agg_sh.at[pl.ds(r0 + zi * ZR, ZR)])
    plsc.subcore_barrier()

    def _wait_idx(j, q):
        pltpu.make_async_copy(src3_hbm.at[w, j], sidx.at[q], sem_i[q]).wait()
        pltpu.make_async_copy(dst3_hbm.at[w, j], didx.at[q], sem_i[q]).wait()

    for b in range(NBUF):
        _wait_idx(b, b)
        pltpu.async_copy(hp_hbm.at[sidx.at[b]], rows.at[b], sem_g[b])

    def outer(jo, _):
        for u in range(RING):
            j = jo * RING + u
            b = u % NBUF
            u5 = (u + NBUF) % RING
            pltpu.make_async_copy(hp_hbm.at[sidx.at[u]],
                                  rows.at[b], sem_g[b]).wait()
            pltpu.sync_copy(rows.at[b], agg_sh.at[didx.at[u]], add=True)
            jr = j + RING

            @pl.when(jr < STEPS)
            def _prefetch_idx():
                pltpu.async_copy(src3_hbm.at[w, jr], sidx.at[u], sem_i[u])
                pltpu.async_copy(dst3_hbm.at[w, jr], didx.at[u], sem_i[u])
            jn = j + NBUF

            @pl.when(jn < STEPS)
            def _issue_gather():
                _wait_idx(jn, u5)
                pltpu.async_copy(hp_hbm.at[sidx.at[u5]], rows.at[b], sem_g[b])
        return 0
    lax.fori_loop(0, OUTER, outer, 0)

    plsc.subcore_barrier()
    pltpu.sync_copy(agg_sh.at[pl.ds(r0, RPT)], out_hbm.at[c, pl.ds(r0, RPT)])


def _prep_body(deg_ref, x_ref, w_ref, dinv_ref, hp_ref):
    deg = deg_ref[0, :, 0:1] + deg_ref[1, :, 0:1] + 1.0
    dinv = lax.rsqrt(deg)
    dinv_ref[...] = dinv
    hp_ref[...] = jnp.dot(x_ref[...], w_ref[...],
                          preferred_element_type=jnp.float32) * dinv


def _tc_prep(deg2, x, w):
    grid = NPAD // BLK
    return pl.pallas_call(
        _prep_body,
        grid=(grid,),
        in_specs=[
            pl.BlockSpec((NC, BLK, DEGW), lambda i: (0, i, 0)),
            pl.BlockSpec((BLK, F), lambda i: (i, 0)),
            pl.BlockSpec((F, F), lambda i: (0, 0)),
        ],
        out_specs=[
            pl.BlockSpec((BLK, 1), lambda i: (i, 0)),
            pl.BlockSpec((BLK, F), lambda i: (i, 0)),
        ],
        out_shape=[
            jax.ShapeDtypeStruct((NPAD, 1), jnp.float32),
            jax.ShapeDtypeStruct((NPAD, F), jnp.float32),
        ],
    )(deg2, x, w)


def _layer_body(a2_ref, hp_ref, dinv_ref, b_ref, w_ref, out_ref):
    dinv = dinv_ref[...]
    conv = (a2_ref[0] + a2_ref[1] + hp_ref[...]) * dinv + b_ref[...]
    h = jnp.tanh(conv)
    out_ref[...] = jnp.dot(h, w_ref[...],
                           preferred_element_type=jnp.float32) * dinv


def _tc_layer(agg2, hp, dinv, b, w):
    grid = NPAD // BLK
    return pl.pallas_call(
        _layer_body,
        grid=(grid,),
        in_specs=[
            pl.BlockSpec((NC, BLK, F), lambda i: (0, i, 0)),
            pl.BlockSpec((BLK, F), lambda i: (i, 0)),
            pl.BlockSpec((BLK, 1), lambda i: (i, 0)),
            pl.BlockSpec((1, F), lambda i: (0, 0)),
            pl.BlockSpec((F, F), lambda i: (0, 0)),
        ],
        out_specs=pl.BlockSpec((BLK, F), lambda i: (i, 0)),
        out_shape=jax.ShapeDtypeStruct((NPAD, F), jnp.float32),
    )(agg2, hp, dinv, b.reshape(1, F), w)


def _pool_body(a2_ref, hp_ref, dinv_ref, b_ref, bid_ref, z_ref,
               gmp_acc, gap_acc, cnt_acc):
    i = pl.program_id(0)

    @pl.when(i == 0)
    def _init():
        gmp_acc[...] = jnp.full((G, F), -jnp.inf, jnp.float32)
        gap_acc[...] = jnp.zeros((G, F), jnp.float32)
        cnt_acc[...] = jnp.zeros((G, F), jnp.float32)

    conv = (a2_ref[0] + a2_ref[1] + hp_ref[...]) * dinv_ref[...] + b_ref[...]
    h = jnp.tanh(conv)
    bid = bid_ref[...]
    gids = lax.broadcasted_iota(jnp.int32, (1, G), 1)
    mask = (bid == gids).astype(jnp.float32)
    gap_acc[...] += lax.dot_general(
        mask, h, (((0,), (0,)), ((), ())), preferred_element_type=jnp.float32)
    cnt_acc[...] += lax.dot_general(
        mask, jnp.ones_like(h), (((0,), (0,)), ((), ())),
        preferred_element_type=jnp.float32)
    for g in range(G):
        rows = jnp.where(bid == g, h, -jnp.inf)
        gmp_acc[g:g + 1, :] = jnp.maximum(gmp_acc[g:g + 1, :],
                                          jnp.max(rows, axis=0, keepdims=True))

    @pl.when(i == pl.num_programs(0) - 1)
    def _fin():
        gap = gap_acc[...] / jnp.maximum(cnt_acc[...], 1.0)
        z_ref[...] = jnp.concatenate([gmp_acc[...], gap], axis=1)


def _tc_pool(agg2, hp, dinv, b, bid):
    grid = NPAD // BLK
    return pl.pallas_call(
        _pool_body,
        grid=(grid,),
        in_specs=[
            pl.BlockSpec((NC, BLK, F), lambda i: (0, i, 0)),
            pl.BlockSpec((BLK, F), lambda i: (i, 0)),
            pl.BlockSpec((BLK, 1), lambda i: (i, 0)),
            pl.BlockSpec((1, F), lambda i: (0, 0)),
            pl.BlockSpec((BLK, 1), lambda i: (i, 0)),
        ],
        out_specs=pl.BlockSpec((G, 2 * F), lambda i: (0, 0)),
        out_shape=jax.ShapeDtypeStruct((G, 2 * F), jnp.float32),
        scratch_shapes=[
            pltpu.VMEM((G, F), jnp.float32),
            pltpu.VMEM((G, F), jnp.float32),
            pltpu.VMEM((G, F), jnp.float32),
        ],
    )(agg2, hp, dinv, b.reshape(1, F), bid)


def _head_body(z_ref, w1_ref, b1_ref, w2_ref, b2_ref, w3_ref, b3_ref, o_ref):
    z = jax.nn.relu(jnp.dot(z_ref[...], w1_ref[...],
                            preferred_element_type=jnp.float32) + b1_ref[...])
    z = jax.nn.relu(jnp.dot(z, w2_ref[...],
                            preferred_element_type=jnp.float32) + b2_ref[...])
    o_ref[...] = jnp.dot(z, w3_ref[...],
                         preferred_element_type=jnp.float32) + b3_ref[...]


def _tc_head(z, fc1_W, fc1_b, fc2_W, fc2_b, outW_pad, outb_pad):
    return pl.pallas_call(
        _head_body,
        out_shape=jax.ShapeDtypeStruct((G, F), jnp.float32),
    )(z, fc1_W, fc1_b.reshape(1, -1), fc2_W, fc2_b.reshape(1, -1),
      outW_pad, outb_pad)


def kernel(x, edge_index, batch_index, W0, b0, W1, b1, W2, b2, W3, b3,
           fc1_W, fc1_b, fc2_W, fc2_b, out_W, out_b):
    src3 = edge_index[0].reshape(NW, STEPS, CH)
    dst3 = edge_index[1].reshape(NW, STEPS, CH)
    x_pad = jnp.pad(x, ((0, NPAD - N), (0, 0)))
    bid_pad = jnp.pad(batch_index, (0, NPAD - N),
                      constant_values=G).reshape(NPAD, 1)
    outW_pad = jnp.pad(out_W, ((0, 0), (0, F - out_W.shape[1])))
    outb_pad = jnp.pad(out_b, (0, F - out_b.shape[0])).reshape(1, F)

    deg2 = _sc_degree(dst3)
    dinv, hp = _tc_prep(deg2, x_pad, W0)

    for b, w in ((b0, W1), (b1, W2), (b2, W3)):
        agg2 = _sc_aggregate(hp, src3, dst3)
        hp = _tc_layer(agg2, hp, dinv, b, w)

    agg2 = _sc_aggregate(hp, src3, dst3)
    z = _tc_pool(agg2, hp, dinv, b3, bid_pad)
    out = _tc_head(z, fc1_W, fc1_b, fc2_W, fc2_b, outW_pad, outb_pad)
    return out[:, :out_W.shape[1]]


# speedup vs baseline: 28.6315x; 1.0172x over previous
import functools

import jax
import jax.numpy as jnp
from jax import lax
from jax.experimental import pallas as pl
from jax.experimental.pallas import tpu as pltpu
from jax.experimental.pallas import tpu_sc as plsc

N = 10000
E = 320000
F = 128
G = 16
NPAD = 10240
BLK = 1024
NC, NS = 2, 16
NW = NC * NS
PER_W = E // NW
CH = 40
STEPS = PER_W // CH
RPT = NPAD // NS
ZR = 64
DEGW = 16
NBUF = 5
RING = 10
OUTER = STEPS // RING

_mesh = plsc.VectorSubcoreMesh(core_axis_name="c", subcore_axis_name="s")


CHD = 80
STEPSD = PER_W // CHD


@functools.partial(
    pl.kernel,
    out_type=jax.ShapeDtypeStruct((NC, NPAD, DEGW), jnp.float32),
    mesh=_mesh,
    scratch_types=[
        pltpu.VMEM((STEPSD, CHD), jnp.int32),
        pltpu.VMEM((CHD, DEGW), jnp.float32),
        pltpu.VMEM((ZR, DEGW), jnp.float32),
        pltpu.VMEM_SHARED((NPAD, DEGW), jnp.float32),
    ],
)
def _sc_degree(dst3_hbm, out_hbm, didx, ones_v, zbuf, deg_sh):
    c = lax.axis_index("c")
    s = lax.axis_index("s")
    w = c * NS + s

    def fill(r, _):
        zbuf[r, pl.ds(0, 16)] = jnp.zeros((16,), jnp.float32)
        return 0
    lax.fori_loop(0, ZR, fill, 0)

    def fill1(r, _):
        ones_v[r, pl.ds(0, 16)] = jnp.ones((16,), jnp.float32)
        return 0
    lax.fori_loop(0, CHD, fill1, 0)

    pltpu.sync_copy(dst3_hbm.at[w], didx)
    r0 = pl.multiple_of(s * RPT, ZR)
    for zi in range(RPT // ZR):
        pltpu.sync_copy(zbuf, deg_sh.at[pl.ds(r0 + zi * ZR, ZR)])
    plsc.subcore_barrier()

    def step(j, _):
        pltpu.sync_copy(ones_v, deg_sh.at[didx.at[j]], add=True)
        return 0
    lax.fori_loop(0, STEPSD, step, 0)

    plsc.subcore_barrier()
    pltpu.sync_copy(deg_sh.at[pl.ds(r0, RPT)], out_hbm.at[c, pl.ds(r0, RPT)])


@functools.partial(
    pl.kernel,
    out_type=jax.ShapeDtypeStruct((NC, NPAD, F), jnp.float32),
    mesh=_mesh,
    scratch_types=[
        pltpu.VMEM((2, RING, CH), jnp.int32),
        pltpu.VMEM((RING, CH), jnp.int32),
        pltpu.VMEM((RING, CH), jnp.int32),
        pltpu.VMEM((NBUF, CH, F), jnp.float32),
        pltpu.VMEM((ZR, F), jnp.float32),
        pltpu.VMEM_SHARED((NPAD, F), jnp.float32),
    ] + [pltpu.SemaphoreType.DMA] * (NBUF + 2),
)
def _sc_aggregate(hp_hbm, src3_hbm, dst3_hbm, out_hbm,
                  sidx, didx0, didx1, rows, zbuf, agg_sh, *sems):
    sem_g = sems[:NBUF]
    sem_i = sems[NBUF:]
    c = lax.axis_index("c")
    s = lax.axis_index("s")
    w = c * NS + s

    def fill(r, _):
        for i in range(F // 16):
            zbuf[r, pl.ds(i * 16, 16)] = jnp.zeros((16,), jnp.float32)
        return 0
    lax.fori_loop(0, ZR, fill, 0)

    dbuf = (didx0, didx1)

    def _blk_issue(k, sl):
        pltpu.async_copy(src3_hbm.at[w, k], sidx.at[sl], sem_i[sl])
        pltpu.async_copy(dst3_hbm.at[w, k], dbuf[sl], sem_i[sl])

    def _blk_wait(k, sl):
        pltpu.make_async_copy(src3_hbm.at[w, k], sidx.at[sl],
                              sem_i[sl]).wait()
        pltpu.make_async_copy(dst3_hbm.at[w, k], dbuf[sl],
                              sem_i[sl]).wait()

    _blk_issue(0, 0)
    _blk_issue(1, 1)
    r0 = pl.multiple_of(s * RPT, ZR)
    for zi in range(RPT // ZR):
        pltpu.sync_copy(zbuf, agg_sh.at[pl.ds(r0 + zi * ZR, ZR)])
    plsc.subcore_barrier()
    _blk_wait(0, 0)
    _blk_wait(1, 1)

    for b in range(NBUF):
        pltpu.async_copy(hp_hbm.at[sidx.at[0, b]], rows.at[b], sem_g[b])

    def outer(jo, _):
        p = jo % 2
        for u in range(RING):
            b = u % NBUF
            pltpu.make_async_copy(hp_hbm.at[sidx.at[p, u]],
                                  rows.at[b], sem_g[b]).wait()

            @pl.when(p == 0)
            def _sc0():
                pltpu.sync_copy(rows.at[b], agg_sh.at[didx0.at[u]], add=True)

            @pl.when(p == 1)
            def _sc1():
                pltpu.sync_copy(rows.at[b], agg_sh.at[didx1.at[u]], add=True)
            if u == NBUF:
                @pl.when((jo >= 1) & (jo + 1 <= OUTER - 1) & (p == 0))
                def _w0():
                    _blk_wait(jo + 1, 1)

                @pl.when((jo >= 1) & (jo + 1 <= OUTER - 1) & (p == 1))
                def _w1():
                    _blk_wait(jo + 1, 0)
            jn = jo * RING + u + NBUF
            sl = p if u < NBUF else 1 - p
            rn = (u + NBUF) % RING

            @pl.when(jn < STEPS)
            def _issue_gather():
                pltpu.async_copy(hp_hbm.at[sidx.at[sl, rn]],
                                 rows.at[b], sem_g[b])

        @pl.when((jo + 2 <= OUTER - 1) & (p == 0))
        def _p0():
            _blk_issue(jo + 2, 0)

        @pl.when((jo + 2 <= OUTER - 1) & (p == 1))
        def _p1():
            _blk_issue(jo + 2, 1)
        return 0
    lax.fori_loop(0, OUTER, outer, 0)

    plsc.subcore_barrier()
    pltpu.sync_copy(agg_sh.at[pl.ds(r0, RPT)], out_hbm.at[c, pl.ds(r0, RPT)])


def _prep_body(deg_ref, x_ref, w_ref, dinv_ref, hp_ref):
    deg = deg_ref[0, :, 0:1] + deg_ref[1, :, 0:1] + 1.0
    dinv = lax.rsqrt(deg)
    dinv_ref[...] = dinv
    hp_ref[...] = jnp.dot(x_ref[...], w_ref[...],
                          preferred_element_type=jnp.float32) * dinv


def _tc_prep(deg2, x, w):
    grid = NPAD // BLK
    return pl.pallas_call(
        _prep_body,
        grid=(grid,),
        in_specs=[
            pl.BlockSpec((NC, BLK, DEGW), lambda i: (0, i, 0)),
            pl.BlockSpec((BLK, F), lambda i: (i, 0)),
            pl.BlockSpec((F, F), lambda i: (0, 0)),
        ],
        out_specs=[
            pl.BlockSpec((BLK, 1), lambda i: (i, 0)),
            pl.BlockSpec((BLK, F), lambda i: (i, 0)),
        ],
        out_shape=[
            jax.ShapeDtypeStruct((NPAD, 1), jnp.float32),
            jax.ShapeDtypeStruct((NPAD, F), jnp.float32),
        ],
    )(deg2, x, w)


def _agg_sum(a2_ref, hp_ref):
    return a2_ref[0] + a2_ref[1] + hp_ref[...]


def _layer_body(a2_ref, hp_ref, dinv_ref, b_ref, w_ref, out_ref):
    dinv = dinv_ref[...]
    conv = _agg_sum(a2_ref, hp_ref) * dinv + b_ref[...]
    h = jnp.tanh(conv)
    out_ref[...] = jnp.dot(h, w_ref[...],
                           preferred_element_type=jnp.float32) * dinv


def _tc_layer(agg2, hp, dinv, b, w):
    grid = NPAD // BLK
    return pl.pallas_call(
        _layer_body,
        grid=(grid,),
        in_specs=[
            pl.BlockSpec((NC, BLK, F), lambda i: (0, i, 0)),
            pl.BlockSpec((BLK, F), lambda i: (i, 0)),
            pl.BlockSpec((BLK, 1), lambda i: (i, 0)),
            pl.BlockSpec((1, F), lambda i: (0, 0)),
            pl.BlockSpec((F, F), lambda i: (0, 0)),
        ],
        out_specs=pl.BlockSpec((BLK, F), lambda i: (i, 0)),
        out_shape=jax.ShapeDtypeStruct((NPAD, F), jnp.float32),
    )(agg2, hp, dinv, b.reshape(1, F), w)


def _pool_body(a2_ref, hp_ref, dinv_ref, b_ref, bid_ref, z_ref,
               gmp_acc, gap_acc, cnt_acc):
    i = pl.program_id(0)

    @pl.when(i == 0)
    def _init():
        gmp_acc[...] = jnp.full((G, F), -jnp.inf, jnp.float32)
        gap_acc[...] = jnp.zeros((G, F), jnp.float32)
        cnt_acc[...] = jnp.zeros((G, F), jnp.float32)

    conv = _agg_sum(a2_ref, hp_ref) * dinv_ref[...] + b_ref[...]
    h = jnp.tanh(conv)
    bid = bid_ref[...]
    gids = lax.broadcasted_iota(jnp.int32, (1, G), 1)
    mask = (bid == gids).astype(jnp.float32)
    gap_acc[...] += lax.dot_general(
        mask, h, (((0,), (0,)), ((), ())), preferred_element_type=jnp.float32)
    cnt_acc[...] += lax.dot_general(
        mask, jnp.ones_like(h), (((0,), (0,)), ((), ())),
        preferred_element_type=jnp.float32)
    for g in range(G):
        rows = jnp.where(bid == g, h, -jnp.inf)
        gmp_acc[g:g + 1, :] = jnp.maximum(gmp_acc[g:g + 1, :],
                                          jnp.max(rows, axis=0, keepdims=True))

    @pl.when(i == pl.num_programs(0) - 1)
    def _fin():
        gap = gap_acc[...] / jnp.maximum(cnt_acc[...], 1.0)
        z_ref[...] = jnp.concatenate([gmp_acc[...], gap], axis=1)


def _tc_pool(agg2, hp, dinv, b, bid):
    grid = NPAD // BLK
    return pl.pallas_call(
        _pool_body,
        grid=(grid,),
        in_specs=[
            pl.BlockSpec((NC, BLK, F), lambda i: (0, i, 0)),
            pl.BlockSpec((BLK, F), lambda i: (i, 0)),
            pl.BlockSpec((BLK, 1), lambda i: (i, 0)),
            pl.BlockSpec((1, F), lambda i: (0, 0)),
            pl.BlockSpec((BLK, 1), lambda i: (i, 0)),
        ],
        out_specs=pl.BlockSpec((G, 2 * F), lambda i: (0, 0)),
        out_shape=jax.ShapeDtypeStruct((G, 2 * F), jnp.float32),
        scratch_shapes=[
            pltpu.VMEM((G, F), jnp.float32),
            pltpu.VMEM((G, F), jnp.float32),
            pltpu.VMEM((G, F), jnp.float32),
        ],
    )(agg2, hp, dinv, b.reshape(1, F), bid)


def _head_body(z_ref, w1_ref, b1_ref, w2_ref, b2_ref, w3_ref, b3_ref, o_ref):
    z = jax.nn.relu(jnp.dot(z_ref[...], w1_ref[...],
                            preferred_element_type=jnp.float32) + b1_ref[...])
    z = jax.nn.relu(jnp.dot(z, w2_ref[...],
                            preferred_element_type=jnp.float32) + b2_ref[...])
    o_ref[...] = jnp.dot(z, w3_ref[...],
                         preferred_element_type=jnp.float32) + b3_ref[...]


def _tc_head(z, fc1_W, fc1_b, fc2_W, fc2_b, outW_pad, outb_pad):
    return pl.pallas_call(
        _head_body,
        out_shape=jax.ShapeDtypeStruct((G, F), jnp.float32),
    )(z, fc1_W, fc1_b.reshape(1, -1), fc2_W, fc2_b.reshape(1, -1),
      outW_pad, outb_pad)


def kernel(x, edge_index, batch_index, W0, b0, W1, b1, W2, b2, W3, b3,
           fc1_W, fc1_b, fc2_W, fc2_b, out_W, out_b):
    src3 = edge_index[0].reshape(NW, OUTER, RING, CH)
    dst3 = edge_index[1].reshape(NW, OUTER, RING, CH)
    dst3d = edge_index[1].reshape(NW, STEPSD, CHD)
    x_pad = jnp.pad(x, ((0, NPAD - N), (0, 0)))
    bid_pad = jnp.pad(batch_index, (0, NPAD - N),
                      constant_values=G).reshape(NPAD, 1)
    outW_pad = jnp.pad(out_W, ((0, 0), (0, F - out_W.shape[1])))
    outb_pad = jnp.pad(out_b, (0, F - out_b.shape[0])).reshape(1, F)

    deg2 = _sc_degree(dst3d)
    dinv, hp = _tc_prep(deg2, x_pad, W0)

    for b, w in ((b0, W1), (b1, W2), (b2, W3)):
        agg2 = _sc_aggregate(hp, src3, dst3)
        hp = _tc_layer(agg2, hp, dinv, b, w)

    agg2 = _sc_aggregate(hp, src3, dst3)
    z = _tc_pool(agg2, hp, dinv, b3, bid_pad)
    out = _tc_head(z, fc1_W, fc1_b, fc2_W, fc2_b, outW_pad, outb_pad)
    return out[:, :out_W.shape[1]]
